# baseline (speedup 1.0000x reference)
import jax
import jax.numpy as jnp
from jax.experimental import pallas as pl
from jax.experimental.pallas import tpu as pltpu

LRELU = 0.01
BW = 4096


def _lrelu(x):
    return jnp.where(x >= 0, x, LRELU * x)


def _mm(a, b):
    return jax.lax.dot_general(a, b, (((1,), (0,)), ((), ())),
                               preferred_element_type=jnp.float32)


def _mm16(a, b):
    return jax.lax.dot_general(a.astype(jnp.bfloat16), b.astype(jnp.bfloat16),
                               (((1,), (0,)), ((), ())),
                               preferred_element_type=jnp.float32)


def _fused_kernel(x_ref, cl1_w, cl1_b, cl2_1_w, cl2_1_b, cl3_1_w, cl3_1_b,
                  w2all, b2t, w3all, b3t,
                  reg1_w, reg1_b, wr2all, br2t, wr3all, br3,
                  msk1_w, msk1_b, msk2_w, msk2_b, msk3_w, msk3_b,
                  s32_1024, s32_256, ones1t,
                  rid1024, rdiv1024, rid32, rdiv256, rid8,
                  xreal_ref, mask_ref):
    x = x_ref[0]

    m = _lrelu(_mm(msk1_w[:], x) + msk1_b[:])
    m = _lrelu(_mm(msk2_w[:], m) + msk2_b[:])
    m = _lrelu(_mm(msk3_w[:], m) + msk3_b[:])
    mask_ref[...] = m.reshape(mask_ref.shape)

    xc = _lrelu(_mm(cl1_w[:], x) + cl1_b[:])
    x2 = _lrelu(_mm(cl2_1_w[:], xc) + cl2_1_b[:])
    logits1 = _mm(cl3_1_w[:], x2) + cl3_1_b[:]
    inds1 = jnp.argmax(logits1, axis=0).astype(jnp.int32)[None, :]

    one = jnp.bfloat16(1.0)
    zero = jnp.bfloat16(0.0)

    full2 = _mm16(w2all[:], xc)
    i1s = inds1.astype(jnp.int16)
    emask1 = jnp.where(rdiv1024[:] == i1s, one, zero)
    oh1 = jnp.where(rid32[:] == i1s, one, zero)
    sel2 = full2.astype(jnp.bfloat16) * emask1
    h = _lrelu(_mm16(s32_1024[:], sel2) + _mm16(b2t[:], oh1))

    full3 = _mm16(w3all[:], h)
    sel3 = full3.astype(jnp.bfloat16) * emask1
    h2 = _mm16(s32_1024[:], sel3) + _mm16(b3t[:], oh1)
    inds2 = jnp.argmax(h2, axis=0).astype(jnp.int32)[None, :]

    inds = inds1 * 32 + inds2
    inds_r = jnp.clip(inds, 0, 1023)
    inds_super = jnp.clip(inds_r // 128, 0, 7)

    xr = _lrelu(_mm16(reg1_w[:], x) + reg1_b[:])

    fullr2 = _mm16(wr2all[:], xr)
    iss = inds_super.astype(jnp.int16)
    emask_s = jnp.where(rdiv256[:] == iss, one, zero)
    oh_s = jnp.where(rid8[:] == iss, one, zero)
    sels = fullr2.astype(jnp.bfloat16) * emask_s
    xr2 = _lrelu(_mm16(s32_256[:], sels) + _mm16(br2t[:], oh_s))

    fullr3 = _mm16(wr3all[:], xr2) + br3[:]
    ohr = jnp.where(rid1024[:] == inds_r.astype(jnp.int16), one, zero)
    selr = fullr3.astype(jnp.bfloat16) * ohr
    regression = _mm16(ones1t[:], selr)

    x_real = (inds.astype(jnp.float32) + regression) * (1.0 / 1024.0)
    xreal_ref[...] = x_real.reshape(xreal_ref.shape)


@jax.jit
def kernel(x_in, cl1_w, cl1_b, cl2_1_w, cl2_1_b, cl3_1_w, cl3_1_b,
           cl2_2_w, cl2_2_b, cl3_2_w, cl3_2_b,
           reg1_w, reg1_b, reg2_w, reg2_b, reg3_w, reg3_b,
           msk1_w, msk1_b, msk2_w, msk2_b, msk3_w, msk3_b):
    B, C, H, W = x_in.shape
    x3 = x_in.reshape(B, C, H * W)

    w2all = cl2_2_w.transpose(0, 2, 1).reshape(32 * 32, 128)
    w3all = cl3_2_w.transpose(0, 2, 1).reshape(32 * 32, 32)
    wr2all = reg2_w.transpose(0, 2, 1).reshape(8 * 32, 128)
    wr3all = reg3_w.reshape(1024, 32)

    r1024 = jnp.arange(32 * 32, dtype=jnp.int32)
    s32_1024 = (jnp.arange(32, dtype=jnp.int32)[:, None] == (r1024 % 32)[None, :]
                ).astype(jnp.float32)
    s32_256 = s32_1024[:, :256]

    col = lambda v: v.reshape(-1, 1)
    bf = lambda v: v.astype(jnp.bfloat16)
    weights = (cl1_w, col(cl1_b), cl2_1_w, col(cl2_1_b), cl3_1_w, col(cl3_1_b),
               bf(w2all), bf(cl2_2_b.T), bf(w3all), bf(cl3_2_b.T),
               bf(reg1_w), col(reg1_b), bf(wr2all), bf(reg2_b.T), bf(wr3all),
               reg3_b,
               msk1_w, col(msk1_b), msk2_w, col(msk2_b), msk3_w, col(msk3_b),
               bf(s32_1024), bf(s32_256),
               jnp.ones((1, 1024), jnp.bfloat16),
               r1024.astype(jnp.int16).reshape(1024, 1),
               (r1024 // 32).astype(jnp.int16).reshape(1024, 1),
               jnp.arange(32, dtype=jnp.int16).reshape(32, 1),
               (jnp.arange(256, dtype=jnp.int32) // 32).astype(jnp.int16).reshape(256, 1),
               jnp.arange(8, dtype=jnp.int16).reshape(8, 1))

    nw = W * H // BW
    grid = (B, nw)
    rep = lambda shape: pl.BlockSpec(shape, lambda b, w: (0,) * len(shape))
    in_specs = [pl.BlockSpec((1, C, BW), lambda b, w: (b, 0, w))]
    in_specs += [rep(a.shape) for a in weights]
    out_specs = [pl.BlockSpec((1, 1, 1, BW), lambda b, w: (b, 0, 0, w)),
                 pl.BlockSpec((1, 1, 1, BW), lambda b, w: (b, 0, 0, w))]
    out_shape = [jax.ShapeDtypeStruct((B, 1, H, W), jnp.float32),
                 jax.ShapeDtypeStruct((B, 1, H, W), jnp.float32)]

    x_real, mask = pl.pallas_call(
        _fused_kernel,
        grid=grid,
        in_specs=in_specs,
        out_specs=out_specs,
        out_shape=out_shape,
        compiler_params=pltpu.CompilerParams(
            dimension_semantics=("parallel", "parallel")),
    )(x3, *weights)
    return (x_real, mask)


# speedup vs baseline: 1.0724x; 1.0724x over previous
import jax
import jax.numpy as jnp
from jax.experimental import pallas as pl
from jax.experimental.pallas import tpu as pltpu

LRELU = 0.01
BW = 4096


def _lrelu(x):
    return jnp.where(x >= 0, x, LRELU * x)


def _mm(a, b):
    return jax.lax.dot_general(a, b, (((1,), (0,)), ((), ())),
                               preferred_element_type=jnp.float32)


def _mm16(a, b):
    return jax.lax.dot_general(a.astype(jnp.bfloat16), b.astype(jnp.bfloat16),
                               (((1,), (0,)), ((), ())),
                               preferred_element_type=jnp.float32)


def _fused_kernel(x_ref, cl1_w, cl1_b, cl2_1_w, cl2_1_b, cl3_1_w, cl3_1_b,
                  w2all, b2t, w3all, b3t,
                  reg1_w, reg1_b, wr2all, br2t, wr3all, br3,
                  msk1_w, msk1_b, msk2_w, msk2_b, msk3_w, msk3_b,
                  s32_1024, s32_256,
                  xreal_ref, mask_ref):
    x = x_ref[0]

    m = _lrelu(_mm(msk1_w[:], x) + msk1_b[:])
    m = _lrelu(_mm(msk2_w[:], m) + msk2_b[:])
    m = _lrelu(_mm(msk3_w[:], m) + msk3_b[:])
    mask_ref[...] = m.reshape(mask_ref.shape)

    xc = _lrelu(_mm(cl1_w[:], x) + cl1_b[:])
    x2 = _lrelu(_mm(cl2_1_w[:], xc) + cl2_1_b[:])
    logits1 = _mm(cl3_1_w[:], x2) + cl3_1_b[:]
    inds1 = jnp.argmax(logits1, axis=0).astype(jnp.int32)[None, :]

    full2 = _mm16(w2all[:], xc)
    row1024 = jax.lax.broadcasted_iota(jnp.int32, (1024, BW), 0)
    emask1 = (row1024 // 32 == inds1).astype(jnp.float32)
    oh1 = (jax.lax.broadcasted_iota(jnp.int32, (32, BW), 0) == inds1
           ).astype(jnp.float32)
    h = _lrelu(_mm16(s32_1024[:], full2 * emask1) + _mm16(b2t[:], oh1))

    full3 = _mm16(w3all[:], h)
    h2 = _mm16(s32_1024[:], full3 * emask1) + _mm16(b3t[:], oh1)
    inds2 = jnp.argmax(h2, axis=0).astype(jnp.int32)[None, :]

    inds = inds1 * 32 + inds2
    inds_r = jnp.clip(inds, 0, 1023)
    inds_super = jnp.clip(inds_r // 128, 0, 7)

    xr = _lrelu(_mm16(reg1_w[:], x) + reg1_b[:])

    fullr2 = _mm16(wr2all[:], xr)
    row256 = jax.lax.broadcasted_iota(jnp.int32, (256, BW), 0)
    emask_s = (row256 // 32 == inds_super).astype(jnp.float32)
    oh_s = (jax.lax.broadcasted_iota(jnp.int32, (8, BW), 0) == inds_super
            ).astype(jnp.float32)
    xr2 = _lrelu(_mm16(s32_256[:], fullr2 * emask_s) + _mm16(br2t[:], oh_s))

    fullr3 = _mm16(wr3all[:], xr2) + br3[:]
    ohr = (row1024 == inds_r).astype(jnp.float32)
    regression = jnp.sum(fullr3 * ohr, axis=0, keepdims=True)

    x_real = (inds.astype(jnp.float32) + regression) * (1.0 / 1024.0)
    xreal_ref[...] = x_real.reshape(xreal_ref.shape)


@jax.jit
def kernel(x_in, cl1_w, cl1_b, cl2_1_w, cl2_1_b, cl3_1_w, cl3_1_b,
           cl2_2_w, cl2_2_b, cl3_2_w, cl3_2_b,
           reg1_w, reg1_b, reg2_w, reg2_b, reg3_w, reg3_b,
           msk1_w, msk1_b, msk2_w, msk2_b, msk3_w, msk3_b):
    B, C, H, W = x_in.shape
    x3 = x_in.reshape(B, C, H * W)

    w2all = cl2_2_w.transpose(0, 2, 1).reshape(32 * 32, 128)
    w3all = cl3_2_w.transpose(0, 2, 1).reshape(32 * 32, 32)
    wr2all = reg2_w.transpose(0, 2, 1).reshape(8 * 32, 128)
    wr3all = reg3_w.reshape(1024, 32)

    r1024 = jnp.arange(32 * 32, dtype=jnp.int32)
    s32_1024 = (jnp.arange(32, dtype=jnp.int32)[:, None] == (r1024 % 32)[None, :]
                ).astype(jnp.float32)
    s32_256 = s32_1024[:, :256]

    col = lambda v: v.reshape(-1, 1)
    bf = lambda v: v.astype(jnp.bfloat16)
    weights = (cl1_w, col(cl1_b), cl2_1_w, col(cl2_1_b), cl3_1_w, col(cl3_1_b),
               bf(w2all), bf(cl2_2_b.T), bf(w3all), bf(cl3_2_b.T),
               bf(reg1_w), col(reg1_b), bf(wr2all), bf(reg2_b.T), bf(wr3all),
               reg3_b,
               msk1_w, col(msk1_b), msk2_w, col(msk2_b), msk3_w, col(msk3_b),
               bf(s32_1024), bf(s32_256))

    nw = W * H // BW
    grid = (B, nw)
    rep = lambda shape: pl.BlockSpec(shape, lambda b, w: (0,) * len(shape))
    in_specs = [pl.BlockSpec((1, C, BW), lambda b, w: (b, 0, w))]
    in_specs += [rep(a.shape) for a in weights]
    out_specs = [pl.BlockSpec((1, 1, 1, BW), lambda b, w: (b, 0, 0, w)),
                 pl.BlockSpec((1, 1, 1, BW), lambda b, w: (b, 0, 0, w))]
    out_shape = [jax.ShapeDtypeStruct((B, 1, H, W), jnp.float32),
                 jax.ShapeDtypeStruct((B, 1, H, W), jnp.float32)]

    x_real, mask = pl.pallas_call(
        _fused_kernel,
        grid=grid,
        in_specs=in_specs,
        out_specs=out_specs,
        out_shape=out_shape,
        compiler_params=pltpu.CompilerParams(
            dimension_semantics=("parallel", "parallel")),
    )(x3, *weights)
    return (x_real, mask)
